# baseline (speedup 1.0000x reference)
import functools

import jax
import jax.numpy as jnp
from jax import lax
from jax.experimental import pallas as pl
from jax.experimental.pallas import tpu as pltpu
from jax.experimental.pallas import tpu_sc as plsc

D = 128
M = D * D * D
KK = 27
NC = 2
NS = 16
NW = NC * NS
GSLICE = M // NW
KCH = 4000
C = 32


def _grid_build_body(keys_hbm, grid_hbm, gbuf, kbuf, n_points):
    wid = lax.axis_index("s") * NC + lax.axis_index("c")
    lo = wid * GSLICE
    neg1 = jnp.full((16,), -1, dtype=jnp.int32)
    iota = lax.iota(jnp.int32, 16)

    def memset(i, carry):
        gbuf[pl.ds(i * 16, 16)] = neg1
        return carry

    lax.fori_loop(0, GSLICE // 16, memset, 0)

    def chunk(cidx, carry):
        pltpu.sync_copy(keys_hbm.at[pl.ds(cidx * KCH, KCH)], kbuf)

        def vec(j, inner):
            kv = kbuf[pl.ds(j * 16, 16)]
            iv = iota + (cidx * KCH + j * 16)
            m = (kv >= lo) & (kv < lo + GSLICE)
            idx = jnp.clip(kv - lo, 0, GSLICE - 1)
            plsc.store_scatter(gbuf, [idx], iv, mask=m)
            return inner

        lax.fori_loop(0, KCH // 16, vec, 0)
        return carry

    @pl.when(wid == NW - 1)
    def _tail():
        pltpu.sync_copy(gbuf.at[pl.ds(0, 16)], grid_hbm.at[pl.ds(M, 16)])

    lax.fori_loop(0, n_points // KCH, chunk, 0)
    pltpu.sync_copy(gbuf, grid_hbm.at[pl.ds(lo, GSLICE)])


def _gather_body(grid_hbm, cpak_hbm, featsz_hbm, g_hbm,
                 cbuf, nkbuf, gvals, rowbuf, fshared,
                 sem, zrow, np_pad):
    wid = lax.axis_index("s") * NC + lax.axis_index("c")
    pw = np_pad // NW

    @pl.when(lax.axis_index("s") == 0)
    def _stage():
        pltpu.sync_copy(featsz_hbm, fshared)

    plsc.subcore_barrier()

    iota = lax.iota(jnp.int32, 16)

    def chunk(cc, carry):
        base = wid * pw + cc * C
        pltpu.sync_copy(cpak_hbm.at[:, pl.ds(base, C)], cbuf)

        for k in range(KK):
            dx, dy, dz = k // 9 - 1, (k // 3) % 3 - 1, k % 3 - 1

            def nk_vec(j, inner, dx=dx, dy=dy, dz=dz, k=k):
                ncx = cbuf[0, pl.ds(j * 16, 16)] + dx
                ncy = cbuf[1, pl.ds(j * 16, 16)] + dy
                ncz = cbuf[2, pl.ds(j * 16, 16)] + dz
                inb = ((ncx >= 0) & (ncx < D) & (ncy >= 0) & (ncy < D)
                       & (ncz >= 0) & (ncz < D))
                ccx = jnp.clip(ncx, 0, D - 1)
                ccy = jnp.clip(ncy, 0, D - 1)
                ccz = jnp.clip(ncz, 0, D - 1)
                nkey = (ccx * D + ccy) * D + ccz
                slot = (j * 16 + iota) * KK + k
                plsc.store_scatter(nkbuf, [slot],
                                   jnp.where(inb, nkey, jnp.int32(M)))
                return inner

            lax.fori_loop(0, C // 16, nk_vec, 0)

        pltpu.async_copy(grid_hbm.at[nkbuf], gvals, sem).wait()

        def safe_vec(j, inner):
            g = gvals[pl.ds(j * 16, 16)]
            gvals[pl.ds(j * 16, 16)] = jnp.where(g >= 0, g, zrow)
            return inner

        lax.fori_loop(0, KK * C // 16, safe_vec, 0)

        pltpu.async_copy(fshared.at[gvals], rowbuf, sem).wait()

        pltpu.async_copy(rowbuf, g_hbm.at[pl.ds(base * KK, C * KK), :],
                         sem).wait()
        return carry

    lax.fori_loop(0, pw // C, chunk, 0)


def _matmul_body(g_ref, w_ref, b_ref, m_ref, out_ref):
    acc = jnp.dot(g_ref[...], w_ref[...], preferred_element_type=jnp.float32)
    out_ref[...] = (acc + b_ref[...]) * m_ref[...]


def kernel(coords, feats, mask_vals, W, b):
    n = coords.shape[0]
    nin = feats.shape[1]
    nout = W.shape[2]
    np_pad = ((n + NW * C - 1) // (NW * C)) * (NW * C)

    cx = coords[:, 0].astype(jnp.int32)
    cy = coords[:, 1].astype(jnp.int32)
    cz = coords[:, 2].astype(jnp.int32)
    keys = (cx * D + cy) * D + cz

    pad = np_pad - n
    cpak = jnp.stack([jnp.pad(cx, (0, pad)), jnp.pad(cy, (0, pad)),
                      jnp.pad(cz, (0, pad))], axis=0)
    featsz = jnp.concatenate(
        [feats, jnp.zeros((1, nin), dtype=feats.dtype)],
        axis=0).astype(jnp.bfloat16)

    mesh = plsc.VectorSubcoreMesh(core_axis_name="c", subcore_axis_name="s")

    grid_build = functools.partial(
        pl.kernel,
        mesh=mesh,
        out_type=jax.ShapeDtypeStruct((M + 16,), jnp.int32),
        scratch_types=[
            pltpu.VMEM((GSLICE,), jnp.int32),
            pltpu.VMEM((KCH,), jnp.int32),
        ],
        compiler_params=pltpu.CompilerParams(needs_layout_passes=False),
    )(functools.partial(_grid_build_body, n_points=n))
    grid = grid_build(keys)

    gather = functools.partial(
        pl.kernel,
        mesh=mesh,
        out_type=jax.ShapeDtypeStruct((np_pad * KK, nin), jnp.bfloat16),
        scratch_types=[
            pltpu.VMEM((3, C), jnp.int32),
            pltpu.VMEM((KK * C,), jnp.int32),
            pltpu.VMEM((KK * C,), jnp.int32),
            pltpu.VMEM((KK * C, nin), jnp.bfloat16),
            pltpu.VMEM_SHARED((n + 1, nin), jnp.bfloat16),
            pltpu.SemaphoreType.DMA,
        ],
        compiler_params=pltpu.CompilerParams(
            needs_layout_passes=False, use_tc_tiling_on_sc=False),
    )(functools.partial(_gather_body, zrow=jnp.int32(n), np_pad=np_pad))
    G = gather(grid, cpak, featsz)

    G2 = G.reshape(np_pad, KK * nin)
    Wf = W.reshape(KK * nin, nout).astype(jnp.bfloat16)
    b2 = b.reshape(1, nout)
    maskp = jnp.pad(mask_vals, ((0, pad), (0, 0)))

    bn = 2048
    out_full = pl.pallas_call(
        _matmul_body,
        grid=(np_pad // bn,),
        in_specs=[
            pl.BlockSpec((bn, KK * nin), lambda i: (i, 0)),
            pl.BlockSpec((KK * nin, nout), lambda i: (0, 0)),
            pl.BlockSpec((1, nout), lambda i: (0, 0)),
            pl.BlockSpec((bn, nout), lambda i: (i, 0)),
        ],
        out_specs=pl.BlockSpec((bn, nout), lambda i: (i, 0)),
        out_shape=jax.ShapeDtypeStruct((np_pad, nout), jnp.float32),
    )(G2, Wf, b2, maskp)
    return out_full[:n]


# speedup vs baseline: 1.0097x; 1.0097x over previous
import functools

import jax
import jax.numpy as jnp
from jax import lax
from jax.experimental import pallas as pl
from jax.experimental.pallas import tpu as pltpu
from jax.experimental.pallas import tpu_sc as plsc

D = 128
M = D * D * D
KK = 27
NC = 2
NS = 16
NW = NC * NS
GSLICE = M // NW
KCH = 4000
C = 32


def _grid_build_body(keys_hbm, grid_hbm, gbuf, kbuf, n_points):
    wid = lax.axis_index("s") * NC + lax.axis_index("c")
    lo = wid * GSLICE
    neg1 = jnp.full((16,), -1, dtype=jnp.int32)
    iota = lax.iota(jnp.int32, 16)

    def memset(i, carry):
        gbuf[pl.ds(i * 16, 16)] = neg1
        return carry

    lax.fori_loop(0, GSLICE // 16, memset, 0)

    def chunk(cidx, carry):
        pltpu.sync_copy(keys_hbm.at[pl.ds(cidx * KCH, KCH)], kbuf)

        def vec(j, inner):
            kv = kbuf[pl.ds(j * 16, 16)]
            iv = iota + (cidx * KCH + j * 16)
            m = (kv >= lo) & (kv < lo + GSLICE)
            idx = jnp.clip(kv - lo, 0, GSLICE - 1)
            plsc.store_scatter(gbuf, [idx], iv, mask=m)
            return inner

        lax.fori_loop(0, KCH // 16, vec, 0)
        return carry

    @pl.when(wid == NW - 1)
    def _tail():
        pltpu.sync_copy(gbuf.at[pl.ds(0, 16)], grid_hbm.at[pl.ds(M, 16)])

    lax.fori_loop(0, n_points // KCH, chunk, 0)
    pltpu.sync_copy(gbuf, grid_hbm.at[pl.ds(lo, GSLICE)])


def _gather_body(grid_hbm, cpak_hbm, featsz_hbm, g_hbm,
                 cbuf, nkbuf, gvals, rowbuf, fshared,
                 sem_g, sem_r, sem_w, zrow, np_pad):
    wid = lax.axis_index("s") * NC + lax.axis_index("c")
    pw = np_pad // NW

    @pl.when(lax.axis_index("s") == 0)
    def _stage():
        pltpu.sync_copy(featsz_hbm, fshared)

    plsc.subcore_barrier()

    iota = lax.iota(jnp.int32, 16)
    nch = pw // C
    kc = KK * C

    def front(cc, par):
        base = wid * pw + cc * C
        pltpu.sync_copy(cpak_hbm.at[:, pl.ds(base, C)],
                        cbuf.at[pl.ds(par * 3, 3), :])
        for k in range(KK):
            dx, dy, dz = k // 9 - 1, (k // 3) % 3 - 1, k % 3 - 1

            def nk_vec(j, inner, dx=dx, dy=dy, dz=dz, k=k):
                ncx = cbuf[par * 3, pl.ds(j * 16, 16)] + dx
                ncy = cbuf[par * 3 + 1, pl.ds(j * 16, 16)] + dy
                ncz = cbuf[par * 3 + 2, pl.ds(j * 16, 16)] + dz
                inb = ((ncx >= 0) & (ncx < D) & (ncy >= 0) & (ncy < D)
                       & (ncz >= 0) & (ncz < D))
                ccx = jnp.clip(ncx, 0, D - 1)
                ccy = jnp.clip(ncy, 0, D - 1)
                ccz = jnp.clip(ncz, 0, D - 1)
                nkey = (ccx * D + ccy) * D + ccz
                slot = par * kc + (j * 16 + iota) * KK + k
                plsc.store_scatter(nkbuf, [slot],
                                   jnp.where(inb, nkey, jnp.int32(M)))
                return inner

            lax.fori_loop(0, C // 16, nk_vec, 0)
        pltpu.async_copy(grid_hbm.at[nkbuf.at[pl.ds(par * kc, kc)]],
                         gvals.at[pl.ds(par * kc, kc)], sem_g)

    front(0, 0)

    def chunk(cc, carry):
        par = cc & 1

        @pl.when(cc + 1 < nch)
        def _next():
            front(cc + 1, 1 - par)

        pltpu.make_async_copy(grid_hbm.at[pl.ds(0, kc)],
                              gvals.at[pl.ds(par * kc, kc)], sem_g).wait()

        def safe_vec(j, inner):
            g = gvals[pl.ds(par * kc + j * 16, 16)]
            gvals[pl.ds(par * kc + j * 16, 16)] = jnp.where(g >= 0, g, zrow)
            return inner

        lax.fori_loop(0, kc // 16, safe_vec, 0)

        base = wid * pw + cc * C

        @pl.when(cc > 0)
        def _drain_prev():
            pltpu.make_async_copy(
                rowbuf, g_hbm.at[pl.ds(base * KK, C * KK), :], sem_w).wait()

        pltpu.async_copy(fshared.at[gvals.at[pl.ds(par * kc, kc)]],
                         rowbuf, sem_r).wait()
        pltpu.async_copy(rowbuf, g_hbm.at[pl.ds(base * KK, C * KK), :], sem_w)
        return carry

    lax.fori_loop(0, nch, chunk, 0)
    last = wid * pw + (nch - 1) * C
    pltpu.make_async_copy(rowbuf, g_hbm.at[pl.ds(last * KK, C * KK), :],
                          sem_w).wait()


def _matmul_body(g_ref, w_ref, b_ref, m_ref, out_ref):
    acc = jnp.dot(g_ref[...], w_ref[...], preferred_element_type=jnp.float32)
    out_ref[...] = (acc + b_ref[...]) * m_ref[...]


def kernel(coords, feats, mask_vals, W, b):
    n = coords.shape[0]
    nin = feats.shape[1]
    nout = W.shape[2]
    np_pad = ((n + NW * C - 1) // (NW * C)) * (NW * C)

    cx = coords[:, 0].astype(jnp.int32)
    cy = coords[:, 1].astype(jnp.int32)
    cz = coords[:, 2].astype(jnp.int32)
    keys = (cx * D + cy) * D + cz

    pad = np_pad - n
    cpak = jnp.stack([jnp.pad(cx, (0, pad)), jnp.pad(cy, (0, pad)),
                      jnp.pad(cz, (0, pad))], axis=0)
    featsz = jnp.concatenate(
        [feats, jnp.zeros((1, nin), dtype=feats.dtype)],
        axis=0).astype(jnp.bfloat16)

    mesh = plsc.VectorSubcoreMesh(core_axis_name="c", subcore_axis_name="s")

    grid_build = functools.partial(
        pl.kernel,
        mesh=mesh,
        out_type=jax.ShapeDtypeStruct((M + 16,), jnp.int32),
        scratch_types=[
            pltpu.VMEM((GSLICE,), jnp.int32),
            pltpu.VMEM((KCH,), jnp.int32),
        ],
        compiler_params=pltpu.CompilerParams(needs_layout_passes=False),
    )(functools.partial(_grid_build_body, n_points=n))
    grid = grid_build(keys)

    gather = functools.partial(
        pl.kernel,
        mesh=mesh,
        out_type=jax.ShapeDtypeStruct((np_pad * KK, nin), jnp.bfloat16),
        scratch_types=[
            pltpu.VMEM((6, C), jnp.int32),
            pltpu.VMEM((2 * KK * C,), jnp.int32),
            pltpu.VMEM((2 * KK * C,), jnp.int32),
            pltpu.VMEM((KK * C, nin), jnp.bfloat16),
            pltpu.VMEM_SHARED((n + 1, nin), jnp.bfloat16),
            pltpu.SemaphoreType.DMA,
            pltpu.SemaphoreType.DMA,
            pltpu.SemaphoreType.DMA,
        ],
        compiler_params=pltpu.CompilerParams(
            needs_layout_passes=False, use_tc_tiling_on_sc=False),
    )(functools.partial(_gather_body, zrow=jnp.int32(n), np_pad=np_pad))
    G = gather(grid, cpak, featsz)

    G2 = G.reshape(np_pad, KK * nin)
    Wf = W.reshape(KK * nin, nout).astype(jnp.bfloat16)
    b2 = b.reshape(1, nout)
    maskp = jnp.pad(mask_vals, ((0, pad), (0, 0)))

    bn = 2048
    out_full = pl.pallas_call(
        _matmul_body,
        grid=(np_pad // bn,),
        in_specs=[
            pl.BlockSpec((bn, KK * nin), lambda i: (i, 0)),
            pl.BlockSpec((KK * nin, nout), lambda i: (0, 0)),
            pl.BlockSpec((1, nout), lambda i: (0, 0)),
            pl.BlockSpec((bn, nout), lambda i: (i, 0)),
        ],
        out_specs=pl.BlockSpec((bn, nout), lambda i: (i, 0)),
        out_shape=jax.ShapeDtypeStruct((np_pad, nout), jnp.float32),
    )(G2, Wf, b2, maskp)
    return out_full[:n]
